# baseline (speedup 1.0000x reference)
import jax
import jax.numpy as jnp
from jax import lax
from jax.experimental import pallas as pl
from jax.experimental.pallas import tpu as pltpu
from jax.experimental.pallas import tpu_sc as plsc

_NC = 2
_NS = 16
_NW = _NC * _NS
_L = 16

_M, _N = 16384, 200
_ROWS_PER_W = _M // _NW
_R = 64
_NCHUNK = _ROWS_PER_W // _R
_COLS = tuple(range(0, _N - _L + 1, _L)) + (_N - _L,)


_GATHER_DNUMS = lax.GatherDimensionNumbers(
    offset_dims=(), collapsed_slice_dims=(0,), start_index_map=(0,))


def _lookup(tbl, idx):
    return lax.gather(tbl, idx[:, None], dimension_numbers=_GATHER_DNUMS,
                      slice_sizes=(1,),
                      mode=lax.GatherScatterMode.PROMISE_IN_BOUNDS)


def _body(tbl_hbm, idx_hbm, out_hbm, tbl_v, idx_v, out_v):
    wid = lax.axis_index("s") * _NC + lax.axis_index("c")
    pltpu.sync_copy(tbl_hbm, tbl_v)
    tbl = tbl_v[...]
    row0 = wid * _ROWS_PER_W

    def chunk(ch, carry):
        base = row0 + ch * _R
        pltpu.sync_copy(idx_hbm.at[pl.ds(base, _R)], idx_v)

        def row(r, carry2):
            for col in _COLS:
                idx = idx_v[r, pl.ds(col, _L)]
                out_v[r, pl.ds(col, _L)] = _lookup(tbl, idx)
            return carry2

        lax.fori_loop(0, _R, row, 0)
        pltpu.sync_copy(out_v, out_hbm.at[pl.ds(base, _R)])
        return carry

    lax.fori_loop(0, _NCHUNK, chunk, 0)


def kernel(inputs, values):
    table16 = jnp.concatenate(
        [values.astype(jnp.float32),
         jnp.full((_L - values.shape[0],), 63.0, jnp.float32)])
    fn = pl.kernel(
        _body,
        out_type=jax.ShapeDtypeStruct((_M, _N), jnp.float32),
        mesh=plsc.VectorSubcoreMesh(
            core_axis_name="c", subcore_axis_name="s", num_cores=_NC),
        scratch_types=[
            pltpu.VMEM((_L,), jnp.float32),
            pltpu.VMEM((_R, _N), jnp.int32),
            pltpu.VMEM((_R, _N), jnp.float32),
        ],
    )
    return fn(table16, inputs)


# speedup vs baseline: 1.0036x; 1.0036x over previous
import jax
import jax.numpy as jnp
from jax import lax
from jax.experimental import pallas as pl
from jax.experimental.pallas import tpu as pltpu
from jax.experimental.pallas import tpu_sc as plsc

_NC = 2
_NS = 16
_NW = _NC * _NS
_L = 16

_M, _N = 16384, 200
_ROWS_PER_W = _M // _NW
_R = 64
_NCHUNK = _ROWS_PER_W // _R
_COLS = tuple(range(0, _N - _L + 1, _L)) + (_N - _L,)


_GATHER_DNUMS = lax.GatherDimensionNumbers(
    offset_dims=(), collapsed_slice_dims=(0,), start_index_map=(0,))


def _lookup(tbl, idx):
    return lax.gather(tbl, idx[:, None], dimension_numbers=_GATHER_DNUMS,
                      slice_sizes=(1,),
                      mode=lax.GatherScatterMode.PROMISE_IN_BOUNDS)


def _body(tbl_hbm, idx_hbm, out_hbm, tbl_v, idx_v, out_v):
    wid = lax.axis_index("s") * _NC + lax.axis_index("c")
    pltpu.sync_copy(tbl_hbm, tbl_v)
    tbl = tbl_v[...]
    row0 = wid * _ROWS_PER_W

    def chunk(ch, carry):
        base = row0 + ch * _R
        pltpu.sync_copy(idx_hbm.at[pl.ds(base, _R)], idx_v)

        @plsc.parallel_loop(0, _R, step=1, unroll=2)
        def row(r):
            for col in _COLS:
                idx = idx_v[r, pl.ds(col, _L)]
                out_v[r, pl.ds(col, _L)] = _lookup(tbl, idx)
        pltpu.sync_copy(out_v, out_hbm.at[pl.ds(base, _R)])
        return carry

    lax.fori_loop(0, _NCHUNK, chunk, 0)


def kernel(inputs, values):
    table16 = jnp.concatenate(
        [values.astype(jnp.float32),
         jnp.full((_L - values.shape[0],), 63.0, jnp.float32)])
    fn = pl.kernel(
        _body,
        out_type=jax.ShapeDtypeStruct((_M, _N), jnp.float32),
        mesh=plsc.VectorSubcoreMesh(
            core_axis_name="c", subcore_axis_name="s", num_cores=_NC),
        scratch_types=[
            pltpu.VMEM((_L,), jnp.float32),
            pltpu.VMEM((_R, _N), jnp.int32),
            pltpu.VMEM((_R, _N), jnp.float32),
        ],
    )
    return fn(table16, inputs)
